# baseline (speedup 1.0000x reference)
import functools

import jax
import jax.numpy as jnp
from jax import lax
from jax.experimental import pallas as pl
from jax.experimental.pallas import tpu as pltpu
from jax.experimental.pallas import tpu_sc as plsc

N = 10000
E = 320000
D = 128
BETA = 0.1

NC = 2
NS = 16
NW = NC * NS

NPAD = 10240
SLICE = NPAD // NS
CHUNK = 128
CPW = 80
EPAD = NW * CPW * CHUNK
ZROWS = 8
HALF = CPW // 2

RB = 1000

def _degs_sc_body(dst_hbm, out_hbm, acc, idx_v, ones_v, z_v):
    c = lax.axis_index("c")
    s = lax.axis_index("s")
    wid = s * NC + c

    for i in range(CHUNK // 16):
        ones_v[pl.ds(i * 16, 16)] = jnp.ones((16,), jnp.float32)

    def zb(i, carry):
        z_v[pl.ds(i * 16, 16)] = jnp.zeros((16,), jnp.float32)
        return carry

    lax.fori_loop(0, SLICE // 16, zb, 0)
    pltpu.sync_copy(z_v, acc.at[pl.ds(s * SLICE, SLICE)])
    pltpu.sync_copy(dst_hbm.at[wid], idx_v)
    plsc.subcore_barrier()

    def body(j, carry):
        pltpu.sync_copy(ones_v, acc.at[idx_v.at[j]], add=True)
        return carry

    lax.fori_loop(0, CPW, body, 0)
    plsc.subcore_barrier()
    pltpu.sync_copy(acc.at[pl.ds(s * SLICE, SLICE)],
                    out_hbm.at[c, pl.ds(s * SLICE, SLICE)])


def _agg_sc_body(h_hbm, src_hbm, dst_hbm, out_hbm, acc, src_v, dst_v, rows_v,
                 z_v, sem0, sem1, ssem0, ssem1):
    c = lax.axis_index("c")
    s = lax.axis_index("s")
    wid = s * NC + c

    def zb(i, carry):
        r = i // (D // 16)
        k = i % (D // 16)
        z_v[r, pl.ds(k * 16, 16)] = jnp.zeros((16,), jnp.float32)
        return carry

    lax.fori_loop(0, ZROWS * (D // 16), zb, 0)

    def zc(t, carry):
        pltpu.sync_copy(z_v, acc.at[pl.ds(s * SLICE + t * ZROWS, ZROWS)])
        return carry

    lax.fori_loop(0, SLICE // ZROWS, zc, 0)
    plsc.subcore_barrier()

    def half_body(half, carry):
        base = half * HALF
        pltpu.sync_copy(src_hbm.at[wid, pl.ds(base, HALF)], src_v)
        pltpu.sync_copy(dst_hbm.at[wid, pl.ds(base, HALF)], dst_v)
        pltpu.async_copy(h_hbm.at[src_v.at[0]], rows_v.at[0], sem0)
        pltpu.async_copy(h_hbm.at[src_v.at[1]], rows_v.at[1], sem1)

        def body(jj, c2):
            j0 = 2 * jj
            pltpu.make_async_copy(h_hbm.at[src_v.at[j0]], rows_v.at[0],
                                  sem0).wait()
            pltpu.async_copy(rows_v.at[0], acc.at[dst_v.at[j0]], ssem0,
                             add=True)
            pltpu.make_async_copy(h_hbm.at[src_v.at[j0 + 1]], rows_v.at[1],
                                  sem1).wait()
            pltpu.async_copy(rows_v.at[1], acc.at[dst_v.at[j0 + 1]], ssem1,
                             add=True)
            pltpu.make_async_copy(rows_v.at[0], acc.at[dst_v.at[j0]],
                                  ssem0).wait()
            pltpu.make_async_copy(rows_v.at[1], acc.at[dst_v.at[j0 + 1]],
                                  ssem1).wait()

            @pl.when(jj + 1 < HALF // 2)
            def _():
                pltpu.async_copy(h_hbm.at[src_v.at[j0 + 2]], rows_v.at[0],
                                 sem0)
                pltpu.async_copy(h_hbm.at[src_v.at[j0 + 3]], rows_v.at[1],
                                 sem1)

            return c2

        lax.fori_loop(0, HALF // 2, body, 0)
        return carry

    lax.fori_loop(0, 2, half_body, 0)
    plsc.subcore_barrier()
    pltpu.sync_copy(acc.at[pl.ds(s * SLICE, SLICE)],
                    out_hbm.at[c, pl.ds(s * SLICE, SLICE)])


@functools.lru_cache(maxsize=None)
def _sc_kernels():
    mesh = plsc.VectorSubcoreMesh(core_axis_name="c", subcore_axis_name="s")
    degs = pl.kernel(
        _degs_sc_body,
        out_type=jax.ShapeDtypeStruct((NC, NPAD), jnp.float32),
        mesh=mesh,
        scratch_types=[
            pltpu.VMEM_SHARED((NPAD,), jnp.float32),
            pltpu.VMEM((CPW, CHUNK), jnp.int32),
            pltpu.VMEM((CHUNK,), jnp.float32),
            pltpu.VMEM((SLICE,), jnp.float32),
        ],
    )
    agg = pl.kernel(
        _agg_sc_body,
        out_type=jax.ShapeDtypeStruct((NC, NPAD, D), jnp.float32),
        mesh=mesh,
        scratch_types=[
            pltpu.VMEM_SHARED((NPAD, D), jnp.float32),
            pltpu.VMEM((HALF, CHUNK), jnp.int32),
            pltpu.VMEM((HALF, CHUNK), jnp.int32),
            pltpu.VMEM((2, CHUNK, D), jnp.float32),
            pltpu.VMEM((ZROWS, D), jnp.float32),
            pltpu.SemaphoreType.DMA,
            pltpu.SemaphoreType.DMA,
            pltpu.SemaphoreType.DMA,
            pltpu.SemaphoreType.DMA,
        ],
    )
    return degs, agg


def _prescale_body(feat_ref, dc_ref, h_ref):
    d = dc_ref[0] + dc_ref[1]
    norm = lax.rsqrt(jnp.maximum(d, 1.0))
    h_ref[...] = feat_ref[...] * norm


_prescale = pl.pallas_call(
    _prescale_body,
    grid=(N // RB,),
    in_specs=[
        pl.BlockSpec((RB, D), lambda i: (i, 0)),
        pl.BlockSpec((NC, RB, 1), lambda i: (0, i, 0)),
    ],
    out_specs=pl.BlockSpec((RB, D), lambda i: (i, 0)),
    out_shape=jax.ShapeDtypeStruct((N, D), jnp.float32),
)


def _dense_body(ap_ref, dc_ref, init_ref, wg1_ref, wg2_ref, bg2_ref, wlin_ref,
                out_ref):
    agg = ap_ref[0] + ap_ref[1]
    d = dc_ref[0] + dc_ref[1]
    norm = lax.rsqrt(jnp.maximum(d, 1.0))
    h2 = agg * norm
    init = init_ref[...]
    z = (jnp.dot(h2, wg1_ref[...], preferred_element_type=jnp.float32)
         + jnp.dot(init, wg2_ref[...], preferred_element_type=jnp.float32)
         + bg2_ref[...])
    scale = jax.nn.sigmoid(z)
    h3 = h2 * scale + init * (1.0 - scale)
    out_ref[...] = (1.0 - BETA) * h3 + BETA * jnp.dot(
        h3, wlin_ref[...], preferred_element_type=jnp.float32)


_dense = pl.pallas_call(
    _dense_body,
    grid=(N // RB,),
    in_specs=[
        pl.BlockSpec((NC, RB, D), lambda i: (0, i, 0)),
        pl.BlockSpec((NC, RB, 1), lambda i: (0, i, 0)),
        pl.BlockSpec((RB, D), lambda i: (i, 0)),
        pl.BlockSpec((D, D), lambda i: (0, 0)),
        pl.BlockSpec((D, D), lambda i: (0, 0)),
        pl.BlockSpec((1, D), lambda i: (0, 0)),
        pl.BlockSpec((D, D), lambda i: (0, 0)),
    ],
    out_specs=pl.BlockSpec((RB, D), lambda i: (i, 0)),
    out_shape=jax.ShapeDtypeStruct((N, D), jnp.float32),
)


def kernel(features, initial_features, edge_index, W_lin, Wg1, Wg2, bg2):
    src = edge_index[0].astype(jnp.int32)
    dst = edge_index[1].astype(jnp.int32)
    pad = EPAD - E
    pad_src = jnp.arange(pad, dtype=jnp.int32) * 37 % N
    pad_dst = N + jnp.arange(pad, dtype=jnp.int32) % (NPAD - N)
    src_p = jnp.concatenate([src, pad_src]).reshape(NW, CPW, CHUNK)
    dst_p = jnp.concatenate([dst, pad_dst]).reshape(NW, CPW, CHUNK)

    _degs_sc, _agg_sc = _sc_kernels()
    degs_p = _degs_sc(dst_p)
    degs_col = degs_p.reshape(NC, NPAD, 1)
    h = _prescale(features, degs_col)
    agg_p = _agg_sc(h, src_p, dst_p)
    return _dense(agg_p, degs_col, initial_features, Wg1.T, Wg2.T,
                  bg2.reshape(1, D), W_lin.T)


# speedup vs baseline: 1.0893x; 1.0893x over previous
import functools

import jax
import jax.numpy as jnp
from jax import lax
from jax.experimental import pallas as pl
from jax.experimental.pallas import tpu as pltpu
from jax.experimental.pallas import tpu_sc as plsc

N = 10000
E = 320000
D = 128
BETA = 0.1

NC = 2
NS = 16
NW = NC * NS

NPAD = 10240
SLICE = NPAD // NS
CHUNK = 128
CPW = 80
EPAD = NW * CPW * CHUNK
ZROWS = 8
HALF = CPW // 2

RB = 1000

def _degs_sc_body(dst_hbm, out_hbm, acc, idx_v, ones_v, z_v):
    c = lax.axis_index("c")
    s = lax.axis_index("s")
    wid = s * NC + c

    for i in range(CHUNK // 16):
        ones_v[pl.ds(i * 16, 16)] = jnp.ones((16,), jnp.float32)

    def zb(i, carry):
        z_v[pl.ds(i * 16, 16)] = jnp.zeros((16,), jnp.float32)
        return carry

    lax.fori_loop(0, SLICE // 16, zb, 0)
    pltpu.sync_copy(z_v, acc.at[pl.ds(s * SLICE, SLICE)])
    pltpu.sync_copy(dst_hbm.at[wid], idx_v)
    plsc.subcore_barrier()

    def body(j, carry):
        pltpu.sync_copy(ones_v, acc.at[idx_v.at[j]], add=True)
        return carry

    lax.fori_loop(0, CPW, body, 0)
    plsc.subcore_barrier()
    pltpu.sync_copy(acc.at[pl.ds(s * SLICE, SLICE)],
                    out_hbm.at[c, pl.ds(s * SLICE, SLICE)])


def _agg_sc_body(h_hbm, src_hbm, dst_hbm, out_hbm, acc, src_v, dst_v, rows_v,
                 z_v, sem0, sem1, ssem0, ssem1):
    c = lax.axis_index("c")
    s = lax.axis_index("s")
    wid = s * NC + c

    def zb(i, carry):
        r = i // (D // 16)
        k = i % (D // 16)
        z_v[r, pl.ds(k * 16, 16)] = jnp.zeros((16,), jnp.float32)
        return carry

    lax.fori_loop(0, ZROWS * (D // 16), zb, 0)

    def zc(t, carry):
        pltpu.sync_copy(z_v, acc.at[pl.ds(s * SLICE + t * ZROWS, ZROWS)])
        return carry

    lax.fori_loop(0, SLICE // ZROWS, zc, 0)
    plsc.subcore_barrier()

    def half_body(half, carry):
        base = half * HALF
        pltpu.sync_copy(src_hbm.at[wid, pl.ds(base, HALF)], src_v)
        pltpu.sync_copy(dst_hbm.at[wid, pl.ds(base, HALF)], dst_v)
        pltpu.async_copy(h_hbm.at[src_v.at[0]], rows_v.at[0], sem0)

        def body(jj, c2):
            j0 = 2 * jj
            pltpu.make_async_copy(h_hbm.at[src_v.at[j0]], rows_v.at[0],
                                  sem0).wait()
            pltpu.async_copy(h_hbm.at[src_v.at[j0 + 1]], rows_v.at[1], sem1)
            pltpu.sync_copy(rows_v.at[0], acc.at[dst_v.at[j0]], add=True)
            pltpu.make_async_copy(h_hbm.at[src_v.at[j0 + 1]], rows_v.at[1],
                                  sem1).wait()

            @pl.when(jj + 1 < HALF // 2)
            def _():
                pltpu.async_copy(h_hbm.at[src_v.at[j0 + 2]], rows_v.at[0],
                                 sem0)

            pltpu.sync_copy(rows_v.at[1], acc.at[dst_v.at[j0 + 1]], add=True)
            return c2

        lax.fori_loop(0, HALF // 2, body, 0)
        return carry

    lax.fori_loop(0, 2, half_body, 0)
    plsc.subcore_barrier()
    pltpu.sync_copy(acc.at[pl.ds(s * SLICE, SLICE)],
                    out_hbm.at[c, pl.ds(s * SLICE, SLICE)])


@functools.lru_cache(maxsize=None)
def _sc_kernels():
    mesh = plsc.VectorSubcoreMesh(core_axis_name="c", subcore_axis_name="s")
    degs = pl.kernel(
        _degs_sc_body,
        out_type=jax.ShapeDtypeStruct((NC, NPAD), jnp.float32),
        mesh=mesh,
        scratch_types=[
            pltpu.VMEM_SHARED((NPAD,), jnp.float32),
            pltpu.VMEM((CPW, CHUNK), jnp.int32),
            pltpu.VMEM((CHUNK,), jnp.float32),
            pltpu.VMEM((SLICE,), jnp.float32),
        ],
    )
    agg = pl.kernel(
        _agg_sc_body,
        out_type=jax.ShapeDtypeStruct((NC, NPAD, D), jnp.float32),
        mesh=mesh,
        scratch_types=[
            pltpu.VMEM_SHARED((NPAD, D), jnp.float32),
            pltpu.VMEM((HALF, CHUNK), jnp.int32),
            pltpu.VMEM((HALF, CHUNK), jnp.int32),
            pltpu.VMEM((2, CHUNK, D), jnp.float32),
            pltpu.VMEM((ZROWS, D), jnp.float32),
            pltpu.SemaphoreType.DMA,
            pltpu.SemaphoreType.DMA,
            pltpu.SemaphoreType.DMA,
            pltpu.SemaphoreType.DMA,
        ],
    )
    return degs, agg


def _prescale_body(feat_ref, dc_ref, h_ref):
    d = dc_ref[0] + dc_ref[1]
    norm = lax.rsqrt(jnp.maximum(d, 1.0))
    h_ref[...] = feat_ref[...] * norm


_prescale = pl.pallas_call(
    _prescale_body,
    grid=(N // RB,),
    in_specs=[
        pl.BlockSpec((RB, D), lambda i: (i, 0)),
        pl.BlockSpec((NC, RB, 1), lambda i: (0, i, 0)),
    ],
    out_specs=pl.BlockSpec((RB, D), lambda i: (i, 0)),
    out_shape=jax.ShapeDtypeStruct((N, D), jnp.float32),
)


def _dense_body(ap_ref, dc_ref, init_ref, wg1_ref, wg2_ref, bg2_ref, wlin_ref,
                out_ref):
    agg = ap_ref[0] + ap_ref[1]
    d = dc_ref[0] + dc_ref[1]
    norm = lax.rsqrt(jnp.maximum(d, 1.0))
    h2 = agg * norm
    init = init_ref[...]
    z = (jnp.dot(h2, wg1_ref[...], preferred_element_type=jnp.float32)
         + jnp.dot(init, wg2_ref[...], preferred_element_type=jnp.float32)
         + bg2_ref[...])
    scale = jax.nn.sigmoid(z)
    h3 = h2 * scale + init * (1.0 - scale)
    out_ref[...] = (1.0 - BETA) * h3 + BETA * jnp.dot(
        h3, wlin_ref[...], preferred_element_type=jnp.float32)


_dense = pl.pallas_call(
    _dense_body,
    grid=(N // RB,),
    in_specs=[
        pl.BlockSpec((NC, RB, D), lambda i: (0, i, 0)),
        pl.BlockSpec((NC, RB, 1), lambda i: (0, i, 0)),
        pl.BlockSpec((RB, D), lambda i: (i, 0)),
        pl.BlockSpec((D, D), lambda i: (0, 0)),
        pl.BlockSpec((D, D), lambda i: (0, 0)),
        pl.BlockSpec((1, D), lambda i: (0, 0)),
        pl.BlockSpec((D, D), lambda i: (0, 0)),
    ],
    out_specs=pl.BlockSpec((RB, D), lambda i: (i, 0)),
    out_shape=jax.ShapeDtypeStruct((N, D), jnp.float32),
)


def kernel(features, initial_features, edge_index, W_lin, Wg1, Wg2, bg2):
    src = edge_index[0].astype(jnp.int32)
    dst = edge_index[1].astype(jnp.int32)
    pad = EPAD - E
    pad_src = jnp.arange(pad, dtype=jnp.int32) * 37 % N
    pad_dst = N + jnp.arange(pad, dtype=jnp.int32) % (NPAD - N)
    src_p = jnp.concatenate([src, pad_src]).reshape(NW, CPW, CHUNK)
    dst_p = jnp.concatenate([dst, pad_dst]).reshape(NW, CPW, CHUNK)

    _degs_sc, _agg_sc = _sc_kernels()
    degs_p = _degs_sc(dst_p)
    degs_col = degs_p.reshape(NC, NPAD, 1)
    h = _prescale(features, degs_col)
    agg_p = _agg_sc(h, src_p, dst_p)
    return _dense(agg_p, degs_col, initial_features, Wg1.T, Wg2.T,
                  bg2.reshape(1, D), W_lin.T)


# speedup vs baseline: 1.2122x; 1.1129x over previous
import functools

import jax
import jax.numpy as jnp
from jax import lax
from jax.experimental import pallas as pl
from jax.experimental.pallas import tpu as pltpu
from jax.experimental.pallas import tpu_sc as plsc

N = 10000
E = 320000
D = 128
BETA = 0.1

NC = 2
NS = 16
NW = NC * NS

NPAD = 10240
SLICE = NPAD // NS
CHUNK = 128
CPW = 80
EPAD = NW * CPW * CHUNK
ZROWS = 8
HALF = CPW // 2

RB = 1000

def _degs_sc_body(dst_hbm, out_hbm, acc, idx_v, ones_v, z_v):
    c = lax.axis_index("c")
    s = lax.axis_index("s")
    wid = s * NC + c

    for i in range(CHUNK // 16):
        ones_v[pl.ds(i * 16, 16)] = jnp.ones((16,), jnp.float32)

    def zb(i, carry):
        z_v[pl.ds(i * 16, 16)] = jnp.zeros((16,), jnp.float32)
        return carry

    lax.fori_loop(0, SLICE // 16, zb, 0)
    pltpu.sync_copy(z_v, acc.at[pl.ds(s * SLICE, SLICE)])
    pltpu.sync_copy(dst_hbm.at[wid], idx_v)
    plsc.subcore_barrier()

    def body(j, carry):
        pltpu.sync_copy(ones_v, acc.at[idx_v.at[j]], add=True)
        return carry

    lax.fori_loop(0, CPW, body, 0)
    plsc.subcore_barrier()
    pltpu.sync_copy(acc.at[pl.ds(s * SLICE, SLICE)],
                    out_hbm.at[c, pl.ds(s * SLICE, SLICE)])


def _agg_sc_body(h_hbm, src_hbm, dst_hbm, out_hbm, acc, src_v, dst_v, rows_v,
                 z_v, sem0, sem1, ssem0, ssem1):
    c = lax.axis_index("c")
    s = lax.axis_index("s")
    wid = s * NC + c

    def zb(i, carry):
        r = i // (D // 16)
        k = i % (D // 16)
        z_v[r, pl.ds(k * 16, 16)] = jnp.zeros((16,), jnp.float32)
        return carry

    lax.fori_loop(0, ZROWS * (D // 16), zb, 0)

    def zc(t, carry):
        pltpu.sync_copy(z_v, acc.at[pl.ds(s * SLICE + t * ZROWS, ZROWS)])
        return carry

    lax.fori_loop(0, SLICE // ZROWS, zc, 0)
    plsc.subcore_barrier()

    def half_body(half, carry):
        base = half * HALF
        pltpu.sync_copy(src_hbm.at[wid, pl.ds(base, HALF)], src_v)
        pltpu.sync_copy(dst_hbm.at[wid, pl.ds(base, HALF)], dst_v)
        pltpu.async_copy(h_hbm.at[src_v.at[0]], rows_v.at[0], sem0)
        pltpu.async_copy(h_hbm.at[src_v.at[1]], rows_v.at[1], sem1)

        def body(jj, c2):
            j0 = 2 * jj
            pltpu.make_async_copy(h_hbm.at[src_v.at[j0]], rows_v.at[0],
                                  sem0).wait()
            pltpu.sync_copy(rows_v.at[0], acc.at[dst_v.at[j0]], add=True)

            @pl.when(jj + 1 < HALF // 2)
            def _():
                pltpu.async_copy(h_hbm.at[src_v.at[j0 + 2]], rows_v.at[0],
                                 sem0)

            pltpu.make_async_copy(h_hbm.at[src_v.at[j0 + 1]], rows_v.at[1],
                                  sem1).wait()
            pltpu.sync_copy(rows_v.at[1], acc.at[dst_v.at[j0 + 1]], add=True)

            @pl.when(jj + 1 < HALF // 2)
            def _():
                pltpu.async_copy(h_hbm.at[src_v.at[j0 + 3]], rows_v.at[1],
                                 sem1)

            return c2

        lax.fori_loop(0, HALF // 2, body, 0)
        return carry

    lax.fori_loop(0, 2, half_body, 0)
    plsc.subcore_barrier()
    pltpu.sync_copy(acc.at[pl.ds(s * SLICE, SLICE)],
                    out_hbm.at[c, pl.ds(s * SLICE, SLICE)])


@functools.lru_cache(maxsize=None)
def _sc_kernels():
    mesh = plsc.VectorSubcoreMesh(core_axis_name="c", subcore_axis_name="s")
    degs = pl.kernel(
        _degs_sc_body,
        out_type=jax.ShapeDtypeStruct((NC, NPAD), jnp.float32),
        mesh=mesh,
        scratch_types=[
            pltpu.VMEM_SHARED((NPAD,), jnp.float32),
            pltpu.VMEM((CPW, CHUNK), jnp.int32),
            pltpu.VMEM((CHUNK,), jnp.float32),
            pltpu.VMEM((SLICE,), jnp.float32),
        ],
    )
    agg = pl.kernel(
        _agg_sc_body,
        out_type=jax.ShapeDtypeStruct((NC, NPAD, D), jnp.float32),
        mesh=mesh,
        scratch_types=[
            pltpu.VMEM_SHARED((NPAD, D), jnp.float32),
            pltpu.VMEM((HALF, CHUNK), jnp.int32),
            pltpu.VMEM((HALF, CHUNK), jnp.int32),
            pltpu.VMEM((2, CHUNK, D), jnp.float32),
            pltpu.VMEM((ZROWS, D), jnp.float32),
            pltpu.SemaphoreType.DMA,
            pltpu.SemaphoreType.DMA,
            pltpu.SemaphoreType.DMA,
            pltpu.SemaphoreType.DMA,
        ],
    )
    return degs, agg


def _prescale_body(feat_ref, dc_ref, h_ref):
    d = dc_ref[0] + dc_ref[1]
    norm = lax.rsqrt(jnp.maximum(d, 1.0))
    h_ref[...] = feat_ref[...] * norm


_prescale = pl.pallas_call(
    _prescale_body,
    grid=(N // RB,),
    in_specs=[
        pl.BlockSpec((RB, D), lambda i: (i, 0)),
        pl.BlockSpec((NC, RB, 1), lambda i: (0, i, 0)),
    ],
    out_specs=pl.BlockSpec((RB, D), lambda i: (i, 0)),
    out_shape=jax.ShapeDtypeStruct((N, D), jnp.float32),
)


def _dense_body(ap_ref, dc_ref, init_ref, wg1_ref, wg2_ref, bg2_ref, wlin_ref,
                out_ref):
    agg = ap_ref[0] + ap_ref[1]
    d = dc_ref[0] + dc_ref[1]
    norm = lax.rsqrt(jnp.maximum(d, 1.0))
    h2 = agg * norm
    init = init_ref[...]
    z = (jnp.dot(h2, wg1_ref[...], preferred_element_type=jnp.float32)
         + jnp.dot(init, wg2_ref[...], preferred_element_type=jnp.float32)
         + bg2_ref[...])
    scale = jax.nn.sigmoid(z)
    h3 = h2 * scale + init * (1.0 - scale)
    out_ref[...] = (1.0 - BETA) * h3 + BETA * jnp.dot(
        h3, wlin_ref[...], preferred_element_type=jnp.float32)


_dense = pl.pallas_call(
    _dense_body,
    grid=(N // RB,),
    in_specs=[
        pl.BlockSpec((NC, RB, D), lambda i: (0, i, 0)),
        pl.BlockSpec((NC, RB, 1), lambda i: (0, i, 0)),
        pl.BlockSpec((RB, D), lambda i: (i, 0)),
        pl.BlockSpec((D, D), lambda i: (0, 0)),
        pl.BlockSpec((D, D), lambda i: (0, 0)),
        pl.BlockSpec((1, D), lambda i: (0, 0)),
        pl.BlockSpec((D, D), lambda i: (0, 0)),
    ],
    out_specs=pl.BlockSpec((RB, D), lambda i: (i, 0)),
    out_shape=jax.ShapeDtypeStruct((N, D), jnp.float32),
)


def kernel(features, initial_features, edge_index, W_lin, Wg1, Wg2, bg2):
    src = edge_index[0].astype(jnp.int32)
    dst = edge_index[1].astype(jnp.int32)
    pad = EPAD - E
    pad_src = jnp.arange(pad, dtype=jnp.int32) * 37 % N
    pad_dst = N + jnp.arange(pad, dtype=jnp.int32) % (NPAD - N)
    src_p = jnp.concatenate([src, pad_src]).reshape(NW, CPW, CHUNK)
    dst_p = jnp.concatenate([dst, pad_dst]).reshape(NW, CPW, CHUNK)

    _degs_sc, _agg_sc = _sc_kernels()
    degs_p = _degs_sc(dst_p)
    degs_col = degs_p.reshape(NC, NPAD, 1)
    h = _prescale(features, degs_col)
    agg_p = _agg_sc(h, src_p, dst_p)
    return _dense(agg_p, degs_col, initial_features, Wg1.T, Wg2.T,
                  bg2.reshape(1, D), W_lin.T)


# speedup vs baseline: 1.2410x; 1.0237x over previous
import functools

import jax
import jax.numpy as jnp
from jax import lax
from jax.experimental import pallas as pl
from jax.experimental.pallas import tpu as pltpu
from jax.experimental.pallas import tpu_sc as plsc

N = 10000
E = 320000
D = 128
BETA = 0.1

NC = 2
NS = 16
NW = NC * NS

NPAD = 10112
SLICE = NPAD // NS
CHUNK = 96
CPW = 108
EPAD = NW * CPW * CHUNK
HALF = CPW // 2

RB = 1000

def _degs_sc_body(dst_hbm, out_hbm, acc, idx_v, ones_v, z_v):
    c = lax.axis_index("c")
    s = lax.axis_index("s")
    wid = s * NC + c

    for i in range(CHUNK // 16):
        ones_v[pl.ds(i * 16, 16)] = jnp.ones((16,), jnp.float32)

    def zb(i, carry):
        z_v[pl.ds(i * 16, 16)] = jnp.zeros((16,), jnp.float32)
        return carry

    lax.fori_loop(0, 40, zb, 0)
    pltpu.sync_copy(z_v.at[pl.ds(0, SLICE)], acc.at[pl.ds(s * SLICE, SLICE)])
    pltpu.sync_copy(dst_hbm.at[wid], idx_v)
    plsc.subcore_barrier()

    def half_body(half, carry):
        def body(j, c2):
            pltpu.sync_copy(ones_v, acc.at[idx_v.at[half, j]], add=True)
            return c2

        lax.fori_loop(0, HALF, body, 0)
        return carry

    lax.fori_loop(0, 2, half_body, 0)
    plsc.subcore_barrier()
    pltpu.sync_copy(acc.at[pl.ds(s * SLICE, SLICE)], z_v.at[pl.ds(0, SLICE)])
    pltpu.sync_copy(z_v.at[pl.ds(0, SLICE)],
                    out_hbm.at[pl.ds(c * NPAD + s * SLICE, SLICE)])


def _agg_sc_body(h_hbm, src_hbm, dst_hbm, out_hbm, acc, src_v, dst_v, rows_v,
                 sem0, sem1, sem2):
    c = lax.axis_index("c")
    s = lax.axis_index("s")
    wid = s * NC + c

    def zb(i, carry):
        r = i // CHUNK
        row = i % CHUNK
        for k in range(D // 16):
            rows_v[r, row, pl.ds(k * 16, 16)] = jnp.zeros((16,), jnp.float32)
        return carry

    lax.fori_loop(0, 3 * CHUNK, zb, 0)
    base = s * SLICE
    for t in range(6):
        pltpu.sync_copy(rows_v.at[0], acc.at[pl.ds(base + t * CHUNK, CHUNK)])
    pltpu.sync_copy(rows_v.at[0, pl.ds(0, 56)],
                    acc.at[pl.ds(base + 6 * CHUNK, 56)])
    plsc.subcore_barrier()

    def half_body(half, carry):
        pltpu.sync_copy(
            src_hbm.at[pl.ds((2 * wid + half) * (HALF * CHUNK),
                             HALF * CHUNK)], src_v)
        pltpu.sync_copy(dst_hbm.at[wid, half], dst_v)
        pltpu.async_copy(h_hbm.at[src_v.at[pl.ds(0, CHUNK)]], rows_v.at[0],
                         sem0)
        pltpu.async_copy(h_hbm.at[src_v.at[pl.ds(CHUNK, CHUNK)]], rows_v.at[1],
                         sem1)
        pltpu.async_copy(h_hbm.at[src_v.at[pl.ds(2 * CHUNK, CHUNK)]],
                         rows_v.at[2], sem2)

        def body(t, c2):
            j = 3 * t
            pltpu.make_async_copy(h_hbm.at[src_v.at[pl.ds(j * CHUNK, CHUNK)]],
                                  rows_v.at[0], sem0).wait()
            pltpu.sync_copy(rows_v.at[0], acc.at[dst_v.at[j]], add=True)

            @pl.when(j + 3 < HALF)
            def _():
                pltpu.async_copy(
                    h_hbm.at[src_v.at[pl.ds((j + 3) * CHUNK, CHUNK)]],
                    rows_v.at[0], sem0)

            pltpu.make_async_copy(
                h_hbm.at[src_v.at[pl.ds((j + 1) * CHUNK, CHUNK)]],
                rows_v.at[1], sem1).wait()
            pltpu.sync_copy(rows_v.at[1], acc.at[dst_v.at[j + 1]], add=True)

            @pl.when(j + 4 < HALF)
            def _():
                pltpu.async_copy(
                    h_hbm.at[src_v.at[pl.ds((j + 4) * CHUNK, CHUNK)]],
                    rows_v.at[1], sem1)

            pltpu.make_async_copy(
                h_hbm.at[src_v.at[pl.ds((j + 2) * CHUNK, CHUNK)]],
                rows_v.at[2], sem2).wait()
            pltpu.sync_copy(rows_v.at[2], acc.at[dst_v.at[j + 2]], add=True)

            @pl.when(j + 5 < HALF)
            def _():
                pltpu.async_copy(
                    h_hbm.at[src_v.at[pl.ds((j + 5) * CHUNK, CHUNK)]],
                    rows_v.at[2], sem2)

            return c2

        lax.fori_loop(0, HALF // 3, body, 0)
        return carry

    lax.fori_loop(0, 2, half_body, 0)
    plsc.subcore_barrier()
    pltpu.sync_copy(acc.at[pl.ds(s * SLICE, SLICE)],
                    out_hbm.at[c, pl.ds(s * SLICE, SLICE)])


@functools.lru_cache(maxsize=None)
def _sc_kernels():
    mesh = plsc.VectorSubcoreMesh(core_axis_name="c", subcore_axis_name="s")
    degs = pl.kernel(
        _degs_sc_body,
        out_type=jax.ShapeDtypeStruct((NC * NPAD,), jnp.float32),
        mesh=mesh,
        scratch_types=[
            pltpu.VMEM_SHARED((NPAD,), jnp.float32),
            pltpu.VMEM((2, HALF, CHUNK), jnp.int32),
            pltpu.VMEM((CHUNK,), jnp.float32),
            pltpu.VMEM((640,), jnp.float32),
        ],
    )
    agg = pl.kernel(
        _agg_sc_body,
        out_type=jax.ShapeDtypeStruct((NC, NPAD, D), jnp.float32),
        mesh=mesh,
        scratch_types=[
            pltpu.VMEM_SHARED((NPAD, D), jnp.float32),
            pltpu.VMEM((HALF * CHUNK,), jnp.int32),
            pltpu.VMEM((HALF, CHUNK), jnp.int32),
            pltpu.VMEM((3, CHUNK, D), jnp.float32),
            pltpu.SemaphoreType.DMA,
            pltpu.SemaphoreType.DMA,
            pltpu.SemaphoreType.DMA,
        ],
    )
    return degs, agg


def _prescale_body(feat_ref, dc_ref, h_ref):
    d = dc_ref[0] + dc_ref[1]
    norm = lax.rsqrt(jnp.maximum(d, 1.0))
    h_ref[...] = feat_ref[...] * norm


_prescale = pl.pallas_call(
    _prescale_body,
    grid=(N // RB,),
    in_specs=[
        pl.BlockSpec((RB, D), lambda i: (i, 0)),
        pl.BlockSpec((NC, RB, 1), lambda i: (0, i, 0)),
    ],
    out_specs=pl.BlockSpec((RB, D), lambda i: (i, 0)),
    out_shape=jax.ShapeDtypeStruct((N, D), jnp.float32),
)


def _dense_body(ap_ref, dc_ref, init_ref, wg1_ref, wg2_ref, bg2_ref, wlin_ref,
                out_ref):
    agg = ap_ref[0] + ap_ref[1]
    d = dc_ref[0] + dc_ref[1]
    norm = lax.rsqrt(jnp.maximum(d, 1.0))
    h2 = agg * norm
    init = init_ref[...]
    z = (jnp.dot(h2, wg1_ref[...], preferred_element_type=jnp.float32)
         + jnp.dot(init, wg2_ref[...], preferred_element_type=jnp.float32)
         + bg2_ref[...])
    scale = jax.nn.sigmoid(z)
    h3 = h2 * scale + init * (1.0 - scale)
    out_ref[...] = (1.0 - BETA) * h3 + BETA * jnp.dot(
        h3, wlin_ref[...], preferred_element_type=jnp.float32)


_dense = pl.pallas_call(
    _dense_body,
    grid=(N // RB,),
    in_specs=[
        pl.BlockSpec((NC, RB, D), lambda i: (0, i, 0)),
        pl.BlockSpec((NC, RB, 1), lambda i: (0, i, 0)),
        pl.BlockSpec((RB, D), lambda i: (i, 0)),
        pl.BlockSpec((D, D), lambda i: (0, 0)),
        pl.BlockSpec((D, D), lambda i: (0, 0)),
        pl.BlockSpec((1, D), lambda i: (0, 0)),
        pl.BlockSpec((D, D), lambda i: (0, 0)),
    ],
    out_specs=pl.BlockSpec((RB, D), lambda i: (i, 0)),
    out_shape=jax.ShapeDtypeStruct((N, D), jnp.float32),
)


def kernel(features, initial_features, edge_index, W_lin, Wg1, Wg2, bg2):
    src = edge_index[0].astype(jnp.int32)
    dst = edge_index[1].astype(jnp.int32)
    pad = EPAD - E
    pad_src = jnp.arange(pad, dtype=jnp.int32) * 37 % N
    pad_dst = N + jnp.arange(pad, dtype=jnp.int32) % (NPAD - N)
    src_p = jnp.concatenate([src, pad_src])
    dst_p = jnp.concatenate([dst, pad_dst]).reshape(NW, 2, HALF, CHUNK)

    _degs_sc, _agg_sc = _sc_kernels()
    degs_p = _degs_sc(dst_p)
    degs_col = degs_p.reshape(NC, NPAD, 1)
    h = _prescale(features, degs_col)
    agg_p = _agg_sc(h, src_p, dst_p)
    return _dense(agg_p, degs_col, initial_features, Wg1.T, Wg2.T,
                  bg2.reshape(1, D), W_lin.T)
